# baseline (speedup 1.0000x reference)
import jax
import jax.numpy as jnp
from jax.experimental import pallas as pl
from jax.experimental.pallas import tpu as pltpu

NUM_CLASSES_ = 1000
ROW_BLOCK = 64
NBUF = 4


def _body(labels_ref, out_hbm, vmem, sems):
    i = pl.program_id(0)
    nsteps = pl.num_programs(0)
    slot = jax.lax.rem(i, NBUF)

    @pl.when(i >= NBUF)
    def _wait_prev():
        pltpu.make_async_copy(
            vmem.at[slot],
            out_hbm.at[pl.ds((i - NBUF) * ROW_BLOCK, ROW_BLOCK)],
            sems.at[slot],
        ).wait()

    labels = labels_ref[...]
    iota = jax.lax.broadcasted_iota(jnp.int32, (1, 1, NUM_CLASSES_), 2)
    vmem[slot] = (labels[:, :, None] == iota).astype(jnp.float32)
    pltpu.make_async_copy(
        vmem.at[slot],
        out_hbm.at[pl.ds(i * ROW_BLOCK, ROW_BLOCK)],
        sems.at[slot],
    ).start()

    @pl.when(i == nsteps - 1)
    def _drain():
        for j in range(NBUF):
            pltpu.make_async_copy(
                vmem.at[j],
                out_hbm.at[pl.ds(j * ROW_BLOCK, ROW_BLOCK)],
                sems.at[j],
            ).wait()


def kernel(labels):
    n, k = labels.shape
    grid = (n // ROW_BLOCK,)
    return pl.pallas_call(
        _body,
        grid=grid,
        in_specs=[pl.BlockSpec((ROW_BLOCK, k), lambda i: (i, 0))],
        out_specs=pl.BlockSpec(memory_space=pl.ANY),
        out_shape=jax.ShapeDtypeStruct((n, k, NUM_CLASSES_), jnp.float32),
        scratch_shapes=[
            pltpu.VMEM((NBUF, ROW_BLOCK, k, NUM_CLASSES_), jnp.float32),
            pltpu.SemaphoreType.DMA((NBUF,)),
        ],
    )(labels)


# speedup vs baseline: 1.0421x; 1.0421x over previous
import jax
import jax.numpy as jnp
from jax.experimental import pallas as pl
from jax.experimental.pallas import tpu as pltpu

N_ROWS = 4096
N_K = 20
N_CLASSES = 1000
ROW_BLOCK = 128
NSLOT = 2

_REGIONS = (
    (0, 16, 0, 896),
    (16, 20, 0, 896),
    (0, 16, 896, 1000),
    (16, 20, 896, 1000),
)


def _body(labels_ref, out_hbm, za, zb, zc, zd, sems):
    i = pl.program_id(0)
    nsteps = pl.num_programs(0)
    slot = jax.lax.rem(i, NSLOT)
    scratches = (za, zb, zc, zd)

    def _copies(s, step):
        base = step * ROW_BLOCK
        return [
            pltpu.make_async_copy(
                scr.at[s],
                out_hbm.at[
                    pl.ds(base, ROW_BLOCK), pl.ds(k0, k1 - k0), pl.ds(c0, c1 - c0)
                ],
                sems.at[s, j],
            )
            for j, ((k0, k1, c0, c1), scr) in enumerate(zip(_REGIONS, scratches))
        ]

    @pl.when(i >= NSLOT)
    def _wait_prev():
        for cp in _copies(slot, i - NSLOT):
            cp.wait()

    labels = labels_ref[...]
    for (k0, k1, c0, c1), scr in zip(_REGIONS, scratches):
        iota = c0 + jax.lax.broadcasted_iota(jnp.int32, (1, 1, c1 - c0), 2)
        scr[slot] = (labels[:, k0:k1, None] == iota).astype(jnp.float32)
    for cp in _copies(slot, i):
        cp.start()

    @pl.when(i == nsteps - 1)
    def _drain():
        for s in range(NSLOT):
            for cp in _copies(s, i):
                cp.wait()


def kernel(labels):
    n, k = labels.shape
    labels_pad = jnp.pad(labels, ((0, 0), (0, 24 - k)), constant_values=-1)
    grid = (n // ROW_BLOCK,)
    return pl.pallas_call(
        _body,
        grid=grid,
        in_specs=[pl.BlockSpec((ROW_BLOCK, 24), lambda i: (i, 0))],
        out_specs=pl.BlockSpec(memory_space=pl.ANY),
        out_shape=jax.ShapeDtypeStruct((n, k, N_CLASSES), jnp.float32),
        scratch_shapes=[
            pltpu.VMEM((NSLOT, ROW_BLOCK, k1 - k0, c1 - c0), jnp.float32)
            for (k0, k1, c0, c1) in _REGIONS
        ] + [pltpu.SemaphoreType.DMA((NSLOT, len(_REGIONS)))],
    )(labels_pad)
